# baseline (speedup 1.0000x reference)
import functools

import jax
import jax.numpy as jnp
from jax import lax
from jax.experimental import pallas as pl
from jax.experimental.pallas import tpu as pltpu
from jax.experimental.pallas import tpu_sc as plsc

_M = 600000
_P1 = 179424941
_P2 = 179425457
_C1 = _P1 % _M
_C2 = _P2 % _M
_C1H = (_C1 * 65536) % _M
_C2H = (_C2 * 65536) % _M

_NC = 2
_NS = 16
_NW = _NC * _NS
_L = 16


@functools.partial(jax.jit, static_argnames=("b", "d"))
def _bloom_lookup(indices_i32, table, *, b, d):
    b_per_w = b // _NW
    n_vec = b_per_w // _L
    mesh = plsc.VectorSubcoreMesh(
        core_axis_name="c", subcore_axis_name="s", num_cores=_NC,
        num_subcores=_NS)

    @functools.partial(
        pl.kernel,
        out_type=jax.ShapeDtypeStruct((b, d), jnp.float32),
        mesh=mesh,
        scratch_types=[
            pltpu.VMEM((b_per_w,), jnp.int32),
            pltpu.VMEM((b_per_w,), jnp.int32),
            pltpu.VMEM((b_per_w,), jnp.int32),
            pltpu.VMEM((b_per_w // 2, d), jnp.float32),
            pltpu.VMEM((b_per_w // 2, d), jnp.float32),
            pltpu.SemaphoreType.DMA,
        ],
        compiler_params=pltpu.CompilerParams(use_tc_tiling_on_sc=False),
    )
    def k(idx_hbm, table_hbm, out_hbm, idx_v, h1_v, h2_v, r1_v, r2_v, sem):
        wid = lax.axis_index("s") * jnp.int32(_NC) + lax.axis_index("c")
        base = wid * jnp.int32(b_per_w)
        pltpu.sync_copy(idx_hbm.at[pl.ds(base, b_per_w)], idx_v)

        def hash_body(k_it, _):
            sl = pl.ds(k_it * jnp.int32(_L), _L)
            i = idx_v[sl]
            hi = lax.shift_right_logical(i, jnp.int32(16))
            lo = lax.bitwise_and(i, jnp.int32(0xFFFF))
            m = jnp.int32(_M)
            h1_v[sl] = (hi * jnp.int32(_C1H) + lo * jnp.int32(_C1)) % m
            h2_v[sl] = (hi * jnp.int32(_C2H) + lo * jnp.int32(_C2)) % m
            return _

        lax.fori_loop(jnp.int32(0), jnp.int32(n_vec), hash_body, None)

        half = b_per_w // 2
        for p in range(2):
            pbase = p * half

            def issue_body(k_it, _):
                off = k_it * jnp.int32(_L)
                v1 = h1_v[pl.ds(jnp.int32(pbase) + off, _L)]
                v2 = h2_v[pl.ds(jnp.int32(pbase) + off, _L)]
                for j in range(_L):
                    pltpu.async_copy(
                        table_hbm.at[pl.ds(v1[j], 1)],
                        r1_v.at[pl.ds(off + j, 1)], sem)
                    pltpu.async_copy(
                        table_hbm.at[pl.ds(v2[j], 1)],
                        r2_v.at[pl.ds(off + j, 1)], sem)
                return _

            lax.fori_loop(jnp.int32(0), jnp.int32(half // _L), issue_body,
                          None)
            pltpu.make_async_copy(
                table_hbm.at[pl.ds(0, half)], r1_v, sem).wait()
            pltpu.make_async_copy(
                table_hbm.at[pl.ds(0, half)], r2_v, sem).wait()

            def avg_body(row, _):
                for cc in range(d // _L):
                    sl = pl.ds(cc * _L, _L)
                    r1_v[row, sl] = (r1_v[row, sl] + r2_v[row, sl]) * 0.5
                return _

            lax.fori_loop(jnp.int32(0), jnp.int32(half), avg_body, None)
            pltpu.sync_copy(
                r1_v, out_hbm.at[pl.ds(base + jnp.int32(pbase), half)])

    return k(indices_i32, table)


def kernel(indices, table):
    b, = indices.shape
    _, d = table.shape
    out = _bloom_lookup(indices.astype(jnp.int32), table, b=b, d=d)
    return out.astype(table.dtype)


# speedup vs baseline: 1.5794x; 1.5794x over previous
import functools

import jax
import jax.numpy as jnp
from jax import lax
from jax.experimental import pallas as pl
from jax.experimental.pallas import tpu as pltpu
from jax.experimental.pallas import tpu_sc as plsc

_M = 600000
_P1 = 179424941
_P2 = 179425457
_C1 = _P1 % _M
_C2 = _P2 % _M
_C1H = (_C1 * 65536) % _M
_C2H = (_C2 * 65536) % _M

_NC = 2
_NS = 16
_NW = _NC * _NS
_L = 16


@functools.partial(jax.jit, static_argnames=("b", "d"))
def _bloom_lookup(indices_i32, table, *, b, d):
    b_per_w = b // _NW
    n_vec = b_per_w // _L
    mesh = plsc.VectorSubcoreMesh(
        core_axis_name="c", subcore_axis_name="s", num_cores=_NC,
        num_subcores=_NS)

    @functools.partial(
        pl.kernel,
        out_type=jax.ShapeDtypeStruct((b, d), jnp.float32),
        mesh=mesh,
        scratch_types=[
            pltpu.VMEM((b_per_w,), jnp.int32),
            pltpu.VMEM((b_per_w,), jnp.int32),
            pltpu.VMEM((b_per_w,), jnp.int32),
            pltpu.VMEM((b_per_w // 2, d), jnp.float32),
            pltpu.VMEM((b_per_w // 2, d), jnp.float32),
            pltpu.SemaphoreType.DMA,
        ],
        compiler_params=pltpu.CompilerParams(use_tc_tiling_on_sc=True),
    )
    def k(idx_hbm, table_hbm, out_hbm, idx_v, h1_v, h2_v, r1_v, r2_v, sem):
        wid = lax.axis_index("s") * jnp.int32(_NC) + lax.axis_index("c")
        base = wid * jnp.int32(b_per_w)
        pltpu.sync_copy(idx_hbm.at[pl.ds(base, b_per_w)], idx_v)

        def hash_body(k_it, _):
            sl = pl.ds(k_it * jnp.int32(_L), _L)
            i = idx_v[sl]
            hi = lax.shift_right_logical(i, jnp.int32(16))
            lo = lax.bitwise_and(i, jnp.int32(0xFFFF))
            m = jnp.int32(_M)
            h1_v[sl] = (hi * jnp.int32(_C1H) + lo * jnp.int32(_C1)) % m
            h2_v[sl] = (hi * jnp.int32(_C2H) + lo * jnp.int32(_C2)) % m
            return _

        lax.fori_loop(jnp.int32(0), jnp.int32(n_vec), hash_body, None)

        half = b_per_w // 2
        for p in range(2):
            pbase = p * half

            def issue_body(k_it, _):
                off = k_it * jnp.int32(_L)
                v1 = h1_v[pl.ds(jnp.int32(pbase) + off, _L)]
                v2 = h2_v[pl.ds(jnp.int32(pbase) + off, _L)]
                for j in range(_L):
                    pltpu.async_copy(
                        table_hbm.at[pl.ds(v1[j], 1)],
                        r1_v.at[pl.ds(off + j, 1)], sem)
                    pltpu.async_copy(
                        table_hbm.at[pl.ds(v2[j], 1)],
                        r2_v.at[pl.ds(off + j, 1)], sem)
                return _

            lax.fori_loop(jnp.int32(0), jnp.int32(half // _L), issue_body,
                          None)
            pltpu.make_async_copy(
                table_hbm.at[pl.ds(0, half)], r1_v, sem).wait()
            pltpu.make_async_copy(
                table_hbm.at[pl.ds(0, half)], r2_v, sem).wait()

            def avg_body(row, _):
                for cc in range(d // _L):
                    sl = pl.ds(cc * _L, _L)
                    r1_v[row, sl] = (r1_v[row, sl] + r2_v[row, sl]) * 0.5
                return _

            lax.fori_loop(jnp.int32(0), jnp.int32(half), avg_body, None)
            pltpu.sync_copy(
                r1_v, out_hbm.at[pl.ds(base + jnp.int32(pbase), half)])

    return k(indices_i32, table)


def kernel(indices, table):
    b, = indices.shape
    _, d = table.shape
    out = _bloom_lookup(indices.astype(jnp.int32), table, b=b, d=d)
    return out.astype(table.dtype)


# speedup vs baseline: 2.5043x; 1.5856x over previous
import functools

import jax
import jax.numpy as jnp
from jax import lax
from jax.experimental import pallas as pl
from jax.experimental.pallas import tpu as pltpu
from jax.experimental.pallas import tpu_sc as plsc

_M = 600000
_P1 = 179424941
_P2 = 179425457
_C1 = _P1 % _M
_C2 = _P2 % _M
_C1H = (_C1 * 65536) % _M
_C2H = (_C2 * 65536) % _M

_NC = 2
_NS = 16
_NW = _NC * _NS
_L = 16


@functools.partial(jax.jit, static_argnames=("b", "d"))
def _bloom_lookup(indices_i32, table, *, b, d):
    b_per_w = b // _NW
    n_vec = b_per_w // _L
    mesh = plsc.VectorSubcoreMesh(
        core_axis_name="c", subcore_axis_name="s", num_cores=_NC,
        num_subcores=_NS)

    @functools.partial(
        pl.kernel,
        out_type=jax.ShapeDtypeStruct((b, d), jnp.float32),
        mesh=mesh,
        scratch_types=[
            pltpu.VMEM((b_per_w,), jnp.int32),
            pltpu.VMEM((b_per_w,), jnp.int32),
            pltpu.VMEM((b_per_w,), jnp.int32),
            pltpu.VMEM((b_per_w // 2, d), jnp.float32),
            pltpu.VMEM((b_per_w // 2, d), jnp.float32),
            pltpu.SemaphoreType.DMA,
        ],
        compiler_params=pltpu.CompilerParams(use_tc_tiling_on_sc=True),
    )
    def k(idx_hbm, table_hbm, out_hbm, idx_v, h1_v, h2_v, r1_v, r2_v, sem):
        wid = lax.axis_index("s") * jnp.int32(_NC) + lax.axis_index("c")
        base = wid * jnp.int32(b_per_w)
        pltpu.sync_copy(idx_hbm.at[pl.ds(base, b_per_w)], idx_v)

        def hash_body(k_it, _):
            sl = pl.ds(k_it * jnp.int32(_L), _L)
            i = idx_v[sl]
            hi = lax.shift_right_logical(i, jnp.int32(16))
            lo = lax.bitwise_and(i, jnp.int32(0xFFFF))
            m = jnp.int32(_M)
            h1_v[sl] = (hi * jnp.int32(_C1H) + lo * jnp.int32(_C1)) % m
            h2_v[sl] = (hi * jnp.int32(_C2H) + lo * jnp.int32(_C2)) % m
            return _

        lax.fori_loop(jnp.int32(0), jnp.int32(n_vec), hash_body, None)

        half = b_per_w // 2
        for p in range(2):
            pbase = p * half

            def issue_body(k_it, _):
                off = k_it * jnp.int32(_L)
                v1 = h1_v[pl.ds(jnp.int32(pbase) + off, _L)]
                v2 = h2_v[pl.ds(jnp.int32(pbase) + off, _L)]
                for j in range(_L):
                    pltpu.async_copy(
                        table_hbm.at[pl.ds(v1[j], 1)],
                        r1_v.at[pl.ds(off + j, 1)], sem)
                    pltpu.async_copy(
                        table_hbm.at[pl.ds(v2[j], 1)],
                        r2_v.at[pl.ds(off + j, 1)], sem)
                return _

            lax.fori_loop(jnp.int32(0), jnp.int32(half // _L), issue_body,
                          None)
            pltpu.make_async_copy(
                table_hbm.at[pl.ds(0, half)], r1_v, sem).wait()
            pltpu.make_async_copy(
                table_hbm.at[pl.ds(0, half)], r2_v, sem).wait()

            def avg_body(row, _):
                for cc in range(d // _L):
                    sl = pl.ds(cc * _L, _L)
                    r1_v[row, sl] = (r1_v[row, sl] + r2_v[row, sl]) * 0.5
                return _

            lax.fori_loop(jnp.int32(0), jnp.int32(half), avg_body, None)
            pltpu.sync_copy(
                r1_v, out_hbm.at[pl.ds(base + jnp.int32(pbase), half)])

    return k(indices_i32, table)


def kernel(indices, table):
    b, = indices.shape
    _, d = table.shape
    table_rm = jnp.transpose(lax.optimization_barrier(table.T))
    out = _bloom_lookup(indices.astype(jnp.int32), table_rm, b=b, d=d)
    return out.astype(table.dtype)
